# Initial kernel scaffold; baseline (speedup 1.0000x reference)
import jax
import jax.numpy as jnp
from jax.experimental import pallas as pl


def kernel(x, edge_index, batch, W1, b1, W2, b2):
    raise NotImplementedError("write your pallas kernel here")


# speedup vs baseline: 5.2004x; 5.2004x over previous
import functools

import jax
import jax.numpy as jnp
from jax import lax
from jax.experimental import pallas as pl
from jax.experimental.pallas import tpu as pltpu
from jax.experimental.pallas import tpu_sc as plsc

N = 10000
E = 320000
D = 128
H = 128
G = 64

NC = 2
NS = 16
NW = NC * NS
E_PER_W = E // NW
CHUNK = 80
N_CHUNKS = E_PER_W // CHUNK
NP_ = 10240
ROWS_PER_TILE = NP_ // NS
ZROWS = 32

BN = 2000


_sc_mesh = plsc.VectorSubcoreMesh(core_axis_name="c", subcore_axis_name="s")


@functools.partial(
    pl.kernel,
    out_type=jax.ShapeDtypeStruct((NC * NP_, H), jnp.float32),
    mesh=_sc_mesh,
    scratch_types=[
        pltpu.VMEM((CHUNK,), jnp.int32),
        pltpu.VMEM((CHUNK,), jnp.int32),
        pltpu.VMEM((CHUNK, H), jnp.float32),
        pltpu.VMEM((ZROWS, H), jnp.float32),
        pltpu.VMEM_SHARED((NP_, H), jnp.float32),
        pltpu.SemaphoreType.DMA,
    ],
)
def _edge_aggregate(y_hbm, src_hbm, dst_hbm, out_hbm,
                    src_v, dst_v, rows_v, zeros_v, acc_sh, sem):
    c = lax.axis_index("c")
    s = lax.axis_index("s")
    wid = s * NC + c

    zvec = jnp.zeros((16,), jnp.float32)

    def zrow(i, carry):
        for j in range(H // 16):
            zeros_v[i, pl.ds(j * 16, 16)] = zvec
        return carry

    lax.fori_loop(0, ZROWS, zrow, 0)

    row0 = s * ROWS_PER_TILE

    def zacc(i, carry):
        pltpu.sync_copy(zeros_v, acc_sh.at[pl.ds(row0 + i * ZROWS, ZROWS)])
        return carry

    lax.fori_loop(0, ROWS_PER_TILE // ZROWS, zacc, 0)

    plsc.subcore_barrier()

    ebase = wid * E_PER_W

    def body(i, carry):
        base = ebase + i * CHUNK
        pltpu.sync_copy(src_hbm.at[pl.ds(base, CHUNK)], src_v)
        pltpu.sync_copy(dst_hbm.at[pl.ds(base, CHUNK)], dst_v)
        pltpu.async_copy(y_hbm.at[src_v], rows_v, sem).wait()
        pltpu.sync_copy(rows_v, acc_sh.at[dst_v], add=True)
        return carry

    lax.fori_loop(0, N_CHUNKS, body, 0)

    plsc.subcore_barrier()

    pltpu.sync_copy(acc_sh.at[pl.ds(row0, ROWS_PER_TILE)],
                    out_hbm.at[pl.ds(c * NP_ + row0, ROWS_PER_TILE)])


def _mm_body(x_ref, w_ref, o_ref):
    o_ref[...] = lax.dot_general(
        x_ref[...], w_ref[...], (((1,), (1,)), ((), ())),
        preferred_element_type=jnp.float32)


def _matmul_wt(x, w):
    return pl.pallas_call(
        _mm_body,
        grid=(N // BN,),
        in_specs=[pl.BlockSpec((BN, D), lambda i: (i, 0)),
                  pl.BlockSpec((H, D), lambda i: (0, 0))],
        out_specs=pl.BlockSpec((BN, H), lambda i: (i, 0)),
        out_shape=jax.ShapeDtypeStruct((N, H), jnp.float32),
    )(x, w)


def _comb_body(p0_ref, p1_ref, b_ref, w_ref, o_ref):
    h = jnp.maximum(p0_ref[...] + p1_ref[...] + b_ref[...], 0.0)
    o_ref[...] = lax.dot_general(
        h, w_ref[...], (((1,), (1,)), ((), ())),
        preferred_element_type=jnp.float32)


def _combine_matmul(p0, p1, b, w):
    return pl.pallas_call(
        _comb_body,
        grid=(N // BN,),
        in_specs=[pl.BlockSpec((BN, H), lambda i: (i, 0)),
                  pl.BlockSpec((BN, H), lambda i: (i, 0)),
                  pl.BlockSpec((1, H), lambda i: (0, 0)),
                  pl.BlockSpec((H, H), lambda i: (0, 0))],
        out_specs=pl.BlockSpec((BN, H), lambda i: (i, 0)),
        out_shape=jax.ShapeDtypeStruct((N, H), jnp.float32),
    )(p0, p1, b, w)


def _pool_body(p0_ref, p1_ref, b_ref, batch_ref, o_ref):
    i = pl.program_id(0)
    h2 = jnp.maximum(p0_ref[...] + p1_ref[...] + b_ref[...], 0.0)
    bb = batch_ref[0, 0, :]
    mask = (lax.broadcasted_iota(jnp.int32, (G, BN), 0)
            == bb[None, :]).astype(jnp.float32)
    acc = lax.dot_general(mask, h2, (((1,), (0,)), ((), ())),
                          preferred_element_type=jnp.float32)

    @pl.when(i == 0)
    def _init():
        o_ref[...] = acc

    @pl.when(i > 0)
    def _accum():
        o_ref[...] += acc


def _pool(p0, p1, b, batch3):
    return pl.pallas_call(
        _pool_body,
        grid=(N // BN,),
        in_specs=[pl.BlockSpec((BN, H), lambda i: (i, 0)),
                  pl.BlockSpec((BN, H), lambda i: (i, 0)),
                  pl.BlockSpec((1, H), lambda i: (0, 0)),
                  pl.BlockSpec((1, 1, BN), lambda i: (i, 0, 0))],
        out_specs=pl.BlockSpec((G, H), lambda i: (0, 0)),
        out_shape=jax.ShapeDtypeStruct((G, H), jnp.float32),
    )(p0, p1, b, batch3)


def kernel(x, edge_index, batch, W1, b1, W2, b2):
    src = edge_index[0]
    dst = edge_index[1]

    y1 = _matmul_wt(x, W1)
    p1 = _edge_aggregate(y1, src, dst)
    y2 = _combine_matmul(p1[:N], p1[NP_:NP_ + N], b1.reshape(1, H), W2)
    p2 = _edge_aggregate(y2, src, dst)
    out = _pool(p2[:N], p2[NP_:NP_ + N], b2.reshape(1, H),
                batch.reshape(N // BN, 1, BN))
    return out


# speedup vs baseline: 9.1729x; 1.7639x over previous
import functools

import jax
import jax.numpy as jnp
from jax import lax
from jax.experimental import pallas as pl
from jax.experimental.pallas import tpu as pltpu
from jax.experimental.pallas import tpu_sc as plsc

N = 10000
E = 320000
D = 128
H = 128
G = 64

NC = 2
NS = 16
NW = NC * NS
E_PER_W = E // NW
CHUNK = 40
N_CHUNKS = E_PER_W // CHUNK
NP_ = 10240
ROWS_PER_TILE = NP_ // NS
ZROWS = 8

BN = 2000


_sc_mesh = plsc.VectorSubcoreMesh(core_axis_name="c", subcore_axis_name="s")


KBUF = 5


@functools.partial(
    pl.kernel,
    out_type=jax.ShapeDtypeStruct((NC * NP_, H), jnp.float32),
    mesh=_sc_mesh,
    scratch_types=[
        pltpu.VMEM((E_PER_W,), jnp.int32),
        pltpu.VMEM((E_PER_W,), jnp.int32),
        [pltpu.VMEM((CHUNK, H), jnp.float32) for _ in range(KBUF)],
        [pltpu.SemaphoreType.DMA for _ in range(KBUF)],
        pltpu.VMEM((ZROWS, H), jnp.float32),
        pltpu.VMEM_SHARED((NP_, H), jnp.float32),
    ],
)
def _edge_aggregate(y_hbm, src_hbm, dst_hbm, out_hbm,
                    src_v, dst_v, rows_bufs, sems, zeros_v, acc_sh):
    c = lax.axis_index("c")
    s = lax.axis_index("s")
    wid = s * NC + c

    pltpu.sync_copy(src_hbm.at[pl.ds(wid * E_PER_W, E_PER_W)], src_v)
    pltpu.sync_copy(dst_hbm.at[pl.ds(wid * E_PER_W, E_PER_W)], dst_v)

    zvec = jnp.zeros((16,), jnp.float32)

    def zrow(i, carry):
        for j in range(H // 16):
            zeros_v[i, pl.ds(j * 16, 16)] = zvec
        return carry

    lax.fori_loop(0, ZROWS, zrow, 0)

    row0 = s * ROWS_PER_TILE

    def zacc(i, carry):
        pltpu.sync_copy(zeros_v, acc_sh.at[pl.ds(row0 + i * ZROWS, ZROWS)])
        return carry

    lax.fori_loop(0, ROWS_PER_TILE // ZROWS, zacc, 0)

    plsc.subcore_barrier()

    def body(k, carry):
        base = k * KBUF * CHUNK
        handles = []
        for b in range(KBUF):
            handles.append(pltpu.async_copy(
                y_hbm.at[src_v.at[pl.ds(base + b * CHUNK, CHUNK)]],
                rows_bufs[b], sems[b]))
        for b in range(KBUF):
            handles[b].wait()
            pltpu.sync_copy(
                rows_bufs[b],
                acc_sh.at[dst_v.at[pl.ds(base + b * CHUNK, CHUNK)]],
                add=True)
        return carry

    lax.fori_loop(0, N_CHUNKS // KBUF, body, 0)

    plsc.subcore_barrier()

    pltpu.sync_copy(acc_sh.at[pl.ds(row0, ROWS_PER_TILE)],
                    out_hbm.at[pl.ds(c * NP_ + row0, ROWS_PER_TILE)])


def _mm_body(x_ref, w_ref, o_ref):
    o_ref[...] = lax.dot_general(
        x_ref[...], w_ref[...], (((1,), (1,)), ((), ())),
        preferred_element_type=jnp.float32)


def _matmul_wt(x, w):
    return pl.pallas_call(
        _mm_body,
        grid=(N // BN,),
        in_specs=[pl.BlockSpec((BN, D), lambda i: (i, 0)),
                  pl.BlockSpec((H, D), lambda i: (0, 0))],
        out_specs=pl.BlockSpec((BN, H), lambda i: (i, 0)),
        out_shape=jax.ShapeDtypeStruct((N, H), jnp.float32),
    )(x, w)


def _comb_body(p0_ref, p1_ref, b_ref, w_ref, o_ref):
    h = jnp.maximum(p0_ref[...] + p1_ref[...] + b_ref[...], 0.0)
    o_ref[...] = lax.dot_general(
        h, w_ref[...], (((1,), (1,)), ((), ())),
        preferred_element_type=jnp.float32)


def _combine_matmul(p0, p1, b, w):
    return pl.pallas_call(
        _comb_body,
        grid=(N // BN,),
        in_specs=[pl.BlockSpec((BN, H), lambda i: (i, 0)),
                  pl.BlockSpec((BN, H), lambda i: (i, 0)),
                  pl.BlockSpec((1, H), lambda i: (0, 0)),
                  pl.BlockSpec((H, H), lambda i: (0, 0))],
        out_specs=pl.BlockSpec((BN, H), lambda i: (i, 0)),
        out_shape=jax.ShapeDtypeStruct((N, H), jnp.float32),
    )(p0, p1, b, w)


def _pool_body(p0_ref, p1_ref, b_ref, batch_ref, o_ref):
    i = pl.program_id(0)
    h2 = jnp.maximum(p0_ref[...] + p1_ref[...] + b_ref[...], 0.0)
    bb = batch_ref[0, 0, :]
    mask = (lax.broadcasted_iota(jnp.int32, (G, BN), 0)
            == bb[None, :]).astype(jnp.float32)
    acc = lax.dot_general(mask, h2, (((1,), (0,)), ((), ())),
                          preferred_element_type=jnp.float32)

    @pl.when(i == 0)
    def _init():
        o_ref[...] = acc

    @pl.when(i > 0)
    def _accum():
        o_ref[...] += acc


def _pool(p0, p1, b, batch3):
    return pl.pallas_call(
        _pool_body,
        grid=(N // BN,),
        in_specs=[pl.BlockSpec((BN, H), lambda i: (i, 0)),
                  pl.BlockSpec((BN, H), lambda i: (i, 0)),
                  pl.BlockSpec((1, H), lambda i: (0, 0)),
                  pl.BlockSpec((1, 1, BN), lambda i: (i, 0, 0))],
        out_specs=pl.BlockSpec((G, H), lambda i: (0, 0)),
        out_shape=jax.ShapeDtypeStruct((G, H), jnp.float32),
    )(p0, p1, b, batch3)


def kernel(x, edge_index, batch, W1, b1, W2, b2):
    src = edge_index[0]
    dst = edge_index[1]

    y1 = _matmul_wt(x, W1)
    p1 = _edge_aggregate(y1, src, dst)
    y2 = _combine_matmul(p1[:N], p1[NP_:NP_ + N], b1.reshape(1, H), W2)
    p2 = _edge_aggregate(y2, src, dst)
    out = _pool(p2[:N], p2[NP_:NP_ + N], b2.reshape(1, H),
                batch.reshape(N // BN, 1, BN))
    return out


# speedup vs baseline: 9.2611x; 1.0096x over previous
import functools

import jax
import jax.numpy as jnp
from jax import lax
from jax.experimental import pallas as pl
from jax.experimental.pallas import tpu as pltpu
from jax.experimental.pallas import tpu_sc as plsc

N = 10000
E = 320000
D = 128
H = 128
G = 64

NC = 2
NS = 16
NW = NC * NS
E_PER_W = E // NW
CHUNK = 40
N_CHUNKS = E_PER_W // CHUNK
NP_ = 10240
ROWS_PER_TILE = NP_ // NS
ZROWS = 8

BN = 10000


_sc_mesh = plsc.VectorSubcoreMesh(core_axis_name="c", subcore_axis_name="s")


KBUF = 5


@functools.partial(
    pl.kernel,
    out_type=jax.ShapeDtypeStruct((NC * NP_, H), jnp.float32),
    mesh=_sc_mesh,
    scratch_types=[
        pltpu.VMEM((E_PER_W,), jnp.int32),
        pltpu.VMEM((E_PER_W,), jnp.int32),
        [pltpu.VMEM((CHUNK, H), jnp.float32) for _ in range(KBUF)],
        [pltpu.SemaphoreType.DMA for _ in range(KBUF)],
        pltpu.VMEM((ZROWS, H), jnp.float32),
        pltpu.VMEM_SHARED((NP_, H), jnp.float32),
    ],
)
def _edge_aggregate(y_hbm, src_hbm, dst_hbm, out_hbm,
                    src_v, dst_v, rows_bufs, sems, zeros_v, acc_sh):
    c = lax.axis_index("c")
    s = lax.axis_index("s")
    wid = s * NC + c

    pltpu.sync_copy(src_hbm.at[pl.ds(wid * E_PER_W, E_PER_W)], src_v)
    pltpu.sync_copy(dst_hbm.at[pl.ds(wid * E_PER_W, E_PER_W)], dst_v)

    zvec = jnp.zeros((16,), jnp.float32)

    def zrow(i, carry):
        for j in range(H // 16):
            zeros_v[i, pl.ds(j * 16, 16)] = zvec
        return carry

    lax.fori_loop(0, ZROWS, zrow, 0)

    row0 = s * ROWS_PER_TILE

    def zacc(i, carry):
        pltpu.sync_copy(zeros_v, acc_sh.at[pl.ds(row0 + i * ZROWS, ZROWS)])
        return carry

    lax.fori_loop(0, ROWS_PER_TILE // ZROWS, zacc, 0)

    plsc.subcore_barrier()

    def body(k, carry):
        base = k * KBUF * CHUNK
        handles = []
        for b in range(KBUF):
            handles.append(pltpu.async_copy(
                y_hbm.at[src_v.at[pl.ds(base + b * CHUNK, CHUNK)]],
                rows_bufs[b], sems[b]))
        for b in range(KBUF):
            handles[b].wait()
            pltpu.sync_copy(
                rows_bufs[b],
                acc_sh.at[dst_v.at[pl.ds(base + b * CHUNK, CHUNK)]],
                add=True)
        return carry

    lax.fori_loop(0, N_CHUNKS // KBUF, body, 0)

    plsc.subcore_barrier()

    pltpu.sync_copy(acc_sh.at[pl.ds(row0, ROWS_PER_TILE)],
                    out_hbm.at[pl.ds(c * NP_ + row0, ROWS_PER_TILE)])


def _mm_body(x_ref, w_ref, o_ref):
    o_ref[...] = lax.dot_general(
        x_ref[...], w_ref[...], (((1,), (1,)), ((), ())),
        preferred_element_type=jnp.float32)


def _matmul_wt(x, w):
    return pl.pallas_call(
        _mm_body,
        grid=(N // BN,),
        in_specs=[pl.BlockSpec((BN, D), lambda i: (i, 0)),
                  pl.BlockSpec((H, D), lambda i: (0, 0))],
        out_specs=pl.BlockSpec((BN, H), lambda i: (i, 0)),
        out_shape=jax.ShapeDtypeStruct((N, H), jnp.float32),
    )(x, w)


def _comb_body(p0_ref, p1_ref, b_ref, w_ref, o_ref):
    h = jnp.maximum(p0_ref[...] + p1_ref[...] + b_ref[...], 0.0)
    o_ref[...] = lax.dot_general(
        h, w_ref[...], (((1,), (1,)), ((), ())),
        preferred_element_type=jnp.float32)


def _combine_matmul(p0, p1, b, w):
    return pl.pallas_call(
        _comb_body,
        grid=(N // BN,),
        in_specs=[pl.BlockSpec((BN, H), lambda i: (i, 0)),
                  pl.BlockSpec((BN, H), lambda i: (i, 0)),
                  pl.BlockSpec((1, H), lambda i: (0, 0)),
                  pl.BlockSpec((H, H), lambda i: (0, 0))],
        out_specs=pl.BlockSpec((BN, H), lambda i: (i, 0)),
        out_shape=jax.ShapeDtypeStruct((N, H), jnp.float32),
    )(p0, p1, b, w)


def _pool_body(p0_ref, p1_ref, b_ref, batch_ref, o_ref):
    i = pl.program_id(0)
    h2 = jnp.maximum(p0_ref[...] + p1_ref[...] + b_ref[...], 0.0)
    bb = batch_ref[0, 0, :]
    mask = (lax.broadcasted_iota(jnp.int32, (G, BN), 0)
            == bb[None, :]).astype(jnp.float32)
    acc = lax.dot_general(mask, h2, (((1,), (0,)), ((), ())),
                          preferred_element_type=jnp.float32)

    @pl.when(i == 0)
    def _init():
        o_ref[...] = acc

    @pl.when(i > 0)
    def _accum():
        o_ref[...] += acc


def _pool(p0, p1, b, batch3):
    return pl.pallas_call(
        _pool_body,
        grid=(N // BN,),
        in_specs=[pl.BlockSpec((BN, H), lambda i: (i, 0)),
                  pl.BlockSpec((BN, H), lambda i: (i, 0)),
                  pl.BlockSpec((1, H), lambda i: (0, 0)),
                  pl.BlockSpec((1, 1, BN), lambda i: (i, 0, 0))],
        out_specs=pl.BlockSpec((G, H), lambda i: (0, 0)),
        out_shape=jax.ShapeDtypeStruct((G, H), jnp.float32),
    )(p0, p1, b, batch3)


def kernel(x, edge_index, batch, W1, b1, W2, b2):
    src = edge_index[0]
    dst = edge_index[1]

    y1 = _matmul_wt(x, W1)
    p1 = _edge_aggregate(y1, src, dst)
    y2 = _combine_matmul(p1[:N], p1[NP_:NP_ + N], b1.reshape(1, H), W2)
    p2 = _edge_aggregate(y2, src, dst)
    out = _pool(p2[:N], p2[NP_:NP_ + N], b2.reshape(1, H),
                batch.reshape(N // BN, 1, BN))
    return out


# speedup vs baseline: 9.4090x; 1.0160x over previous
import functools

import jax
import jax.numpy as jnp
from jax import lax
from jax.experimental import pallas as pl
from jax.experimental.pallas import tpu as pltpu
from jax.experimental.pallas import tpu_sc as plsc

N = 10000
E = 320000
D = 128
H = 128
G = 64

NC = 2
NS = 16
NW = NC * NS
E_PER_W = E // NW
CHUNK = 40
N_CHUNKS = E_PER_W // CHUNK
NP_ = 10240
ROWS_PER_TILE = NP_ // NS
ZROWS = 8

BN = 10000


_sc_mesh = plsc.VectorSubcoreMesh(core_axis_name="c", subcore_axis_name="s")


KBUF = 5


@functools.partial(
    pl.kernel,
    out_type=jax.ShapeDtypeStruct((NC * NP_, H), jnp.float32),
    mesh=_sc_mesh,
    scratch_types=[
        pltpu.VMEM((E_PER_W,), jnp.int32),
        pltpu.VMEM((E_PER_W,), jnp.int32),
        [pltpu.VMEM((CHUNK, H), jnp.float32) for _ in range(KBUF)],
        [pltpu.SemaphoreType.DMA for _ in range(KBUF)],
        pltpu.VMEM((ZROWS, H), jnp.float32),
        pltpu.VMEM_SHARED((NP_, H), jnp.float32),
    ],
)
def _edge_aggregate(y_hbm, src_hbm, dst_hbm, out_hbm,
                    src_v, dst_v, rows_bufs, sems, zeros_v, acc_sh):
    c = lax.axis_index("c")
    s = lax.axis_index("s")
    wid = s * NC + c

    pltpu.sync_copy(src_hbm.at[pl.ds(wid * E_PER_W, E_PER_W)], src_v)
    pltpu.sync_copy(dst_hbm.at[pl.ds(wid * E_PER_W, E_PER_W)], dst_v)

    zvec = jnp.zeros((16,), jnp.float32)

    def zrow(i, carry):
        for j in range(H // 16):
            zeros_v[i, pl.ds(j * 16, 16)] = zvec
        return carry

    lax.fori_loop(0, ZROWS, zrow, 0)

    row0 = s * ROWS_PER_TILE

    def zacc(i, carry):
        pltpu.sync_copy(zeros_v, acc_sh.at[pl.ds(row0 + i * ZROWS, ZROWS)])
        return carry

    lax.fori_loop(0, ROWS_PER_TILE // ZROWS, zacc, 0)

    plsc.subcore_barrier()

    def body(k, carry):
        base = k * KBUF * CHUNK
        handles = []
        for b in range(KBUF):
            handles.append(pltpu.async_copy(
                y_hbm.at[src_v.at[pl.ds(base + b * CHUNK, CHUNK)]],
                rows_bufs[b], sems[b]))
        for b in range(KBUF):
            handles[b].wait()
            pltpu.sync_copy(
                rows_bufs[b],
                acc_sh.at[dst_v.at[pl.ds(base + b * CHUNK, CHUNK)]],
                add=True)
        return carry

    lax.fori_loop(0, N_CHUNKS // KBUF, body, 0)

    plsc.subcore_barrier()

    pltpu.sync_copy(acc_sh.at[pl.ds(row0, ROWS_PER_TILE)],
                    out_hbm.at[pl.ds(c * NP_ + row0, ROWS_PER_TILE)])


def _dbl_body(p0_ref, p1_ref, b_ref, w1_ref, w2_ref, o_ref):
    agg = p0_ref[...] + p1_ref[...]
    h = jnp.maximum(
        lax.dot_general(agg, w1_ref[...], (((1,), (1,)), ((), ())),
                        preferred_element_type=jnp.float32) + b_ref[...],
        0.0)
    o_ref[...] = lax.dot_general(
        h, w2_ref[...], (((1,), (1,)), ((), ())),
        preferred_element_type=jnp.float32)


def _double_matmul(p0, p1, b, w1, w2):
    return pl.pallas_call(
        _dbl_body,
        grid=(N // BN,),
        in_specs=[pl.BlockSpec((BN, H), lambda i: (i, 0)),
                  pl.BlockSpec((BN, H), lambda i: (i, 0)),
                  pl.BlockSpec((1, H), lambda i: (0, 0)),
                  pl.BlockSpec((H, D), lambda i: (0, 0)),
                  pl.BlockSpec((H, H), lambda i: (0, 0))],
        out_specs=pl.BlockSpec((BN, H), lambda i: (i, 0)),
        out_shape=jax.ShapeDtypeStruct((N, H), jnp.float32),
    )(p0, p1, b, w1, w2)


def _pool_body(p0_ref, p1_ref, b_ref, batch_ref, o_ref):
    i = pl.program_id(0)
    h2 = jnp.maximum(p0_ref[...] + p1_ref[...] + b_ref[...], 0.0)
    bb = batch_ref[0, 0, :]
    mask = (lax.broadcasted_iota(jnp.int32, (G, BN), 0)
            == bb[None, :]).astype(jnp.float32)
    acc = lax.dot_general(mask, h2, (((1,), (0,)), ((), ())),
                          preferred_element_type=jnp.float32)

    @pl.when(i == 0)
    def _init():
        o_ref[...] = acc

    @pl.when(i > 0)
    def _accum():
        o_ref[...] += acc


def _pool(p0, p1, b, batch3):
    return pl.pallas_call(
        _pool_body,
        grid=(N // BN,),
        in_specs=[pl.BlockSpec((BN, H), lambda i: (i, 0)),
                  pl.BlockSpec((BN, H), lambda i: (i, 0)),
                  pl.BlockSpec((1, H), lambda i: (0, 0)),
                  pl.BlockSpec((1, 1, BN), lambda i: (i, 0, 0))],
        out_specs=pl.BlockSpec((G, H), lambda i: (0, 0)),
        out_shape=jax.ShapeDtypeStruct((G, H), jnp.float32),
    )(p0, p1, b, batch3)


def kernel(x, edge_index, batch, W1, b1, W2, b2):
    src = edge_index[0]
    dst = edge_index[1]

    p1 = _edge_aggregate(x, src, dst)
    y2 = _double_matmul(p1[:N], p1[NP_:NP_ + N], b1.reshape(1, H), W1, W2)
    p2 = _edge_aggregate(y2, src, dst)
    out = _pool(p2[:N], p2[NP_:NP_ + N], b2.reshape(1, H),
                batch.reshape(N // BN, 1, BN))
    return out


# speedup vs baseline: 10.0741x; 1.0707x over previous
import functools

import jax
import jax.numpy as jnp
from jax import lax
from jax.experimental import pallas as pl
from jax.experimental.pallas import tpu as pltpu
from jax.experimental.pallas import tpu_sc as plsc

N = 10000
E = 320000
D = 128
H = 128
G = 64

NC = 2
NS = 16
NW = NC * NS
E_PER_W = E // NW
CHUNK = 40
N_CHUNKS = E_PER_W // CHUNK
NP_ = 10240
ROWS_PER_TILE = NP_ // NS
ZROWS = 8

BN = 10000


_sc_mesh = plsc.VectorSubcoreMesh(core_axis_name="c", subcore_axis_name="s")


KBUF = 5


@functools.partial(
    pl.kernel,
    out_type=jax.ShapeDtypeStruct((NC * NP_, H), jnp.float32),
    mesh=_sc_mesh,
    scratch_types=[
        pltpu.VMEM((E_PER_W,), jnp.int32),
        pltpu.VMEM((E_PER_W,), jnp.int32),
        [pltpu.VMEM((CHUNK, H), jnp.float32) for _ in range(KBUF)],
        [pltpu.SemaphoreType.DMA for _ in range(KBUF)],
        [pltpu.SemaphoreType.DMA for _ in range(KBUF)],
        pltpu.VMEM((ZROWS, H), jnp.float32),
        pltpu.VMEM_SHARED((NP_, H), jnp.float32),
    ],
)
def _edge_aggregate(y_hbm, src_hbm, dst_hbm, out_hbm,
                    src_v, dst_v, rows_bufs, sems, ssems, zeros_v, acc_sh):
    c = lax.axis_index("c")
    s = lax.axis_index("s")
    wid = s * NC + c

    pltpu.sync_copy(src_hbm.at[pl.ds(wid * E_PER_W, E_PER_W)], src_v)
    pltpu.sync_copy(dst_hbm.at[pl.ds(wid * E_PER_W, E_PER_W)], dst_v)

    zvec = jnp.zeros((16,), jnp.float32)

    def zrow(i, carry):
        for j in range(H // 16):
            zeros_v[i, pl.ds(j * 16, 16)] = zvec
        return carry

    lax.fori_loop(0, ZROWS, zrow, 0)

    row0 = s * ROWS_PER_TILE

    def zacc(i, carry):
        pltpu.sync_copy(zeros_v, acc_sh.at[pl.ds(row0 + i * ZROWS, ZROWS)])
        return carry

    lax.fori_loop(0, ROWS_PER_TILE // ZROWS, zacc, 0)

    plsc.subcore_barrier()

    def body(k, carry):
        base = k * KBUF * CHUNK
        gh = []
        for b in range(KBUF):
            gh.append(pltpu.async_copy(
                y_hbm.at[src_v.at[pl.ds(base + b * CHUNK, CHUNK)]],
                rows_bufs[b], sems[b]))
        sh = []
        for b in range(KBUF):
            gh[b].wait()
            sh.append(pltpu.async_copy(
                rows_bufs[b],
                acc_sh.at[dst_v.at[pl.ds(base + b * CHUNK, CHUNK)]],
                ssems[b], add=True))
        for b in range(KBUF):
            sh[b].wait()
        return carry

    lax.fori_loop(0, N_CHUNKS // KBUF, body, 0)

    plsc.subcore_barrier()

    pltpu.sync_copy(acc_sh.at[pl.ds(row0, ROWS_PER_TILE)],
                    out_hbm.at[pl.ds(c * NP_ + row0, ROWS_PER_TILE)])


def _dbl_body(p0_ref, p1_ref, b_ref, w1_ref, w2_ref, o_ref):
    agg = p0_ref[...] + p1_ref[...]
    h = jnp.maximum(
        lax.dot_general(agg, w1_ref[...], (((1,), (1,)), ((), ())),
                        preferred_element_type=jnp.float32) + b_ref[...],
        0.0)
    o_ref[...] = lax.dot_general(
        h, w2_ref[...], (((1,), (1,)), ((), ())),
        preferred_element_type=jnp.float32)


def _double_matmul(p0, p1, b, w1, w2):
    return pl.pallas_call(
        _dbl_body,
        grid=(N // BN,),
        in_specs=[pl.BlockSpec((BN, H), lambda i: (i, 0)),
                  pl.BlockSpec((BN, H), lambda i: (i, 0)),
                  pl.BlockSpec((1, H), lambda i: (0, 0)),
                  pl.BlockSpec((H, D), lambda i: (0, 0)),
                  pl.BlockSpec((H, H), lambda i: (0, 0))],
        out_specs=pl.BlockSpec((BN, H), lambda i: (i, 0)),
        out_shape=jax.ShapeDtypeStruct((N, H), jnp.float32),
    )(p0, p1, b, w1, w2)


def _pool_body(p0_ref, p1_ref, b_ref, batch_ref, o_ref):
    i = pl.program_id(0)
    h2 = jnp.maximum(p0_ref[...] + p1_ref[...] + b_ref[...], 0.0)
    bb = batch_ref[0, 0, :]
    mask = (lax.broadcasted_iota(jnp.int32, (G, BN), 0)
            == bb[None, :]).astype(jnp.float32)
    acc = lax.dot_general(mask, h2, (((1,), (0,)), ((), ())),
                          preferred_element_type=jnp.float32)

    @pl.when(i == 0)
    def _init():
        o_ref[...] = acc

    @pl.when(i > 0)
    def _accum():
        o_ref[...] += acc


def _pool(p0, p1, b, batch3):
    return pl.pallas_call(
        _pool_body,
        grid=(N // BN,),
        in_specs=[pl.BlockSpec((BN, H), lambda i: (i, 0)),
                  pl.BlockSpec((BN, H), lambda i: (i, 0)),
                  pl.BlockSpec((1, H), lambda i: (0, 0)),
                  pl.BlockSpec((1, 1, BN), lambda i: (i, 0, 0))],
        out_specs=pl.BlockSpec((G, H), lambda i: (0, 0)),
        out_shape=jax.ShapeDtypeStruct((G, H), jnp.float32),
    )(p0, p1, b, batch3)


def kernel(x, edge_index, batch, W1, b1, W2, b2):
    src = edge_index[0]
    dst = edge_index[1]

    p1 = _edge_aggregate(x, src, dst)
    y2 = _double_matmul(p1[:N], p1[NP_:NP_ + N], b1.reshape(1, H), W1, W2)
    p2 = _edge_aggregate(y2, src, dst)
    out = _pool(p2[:N], p2[NP_:NP_ + N], b2.reshape(1, H),
                batch.reshape(N // BN, 1, BN))
    return out


# speedup vs baseline: 10.4263x; 1.0350x over previous
import functools

import jax
import jax.numpy as jnp
from jax import lax
from jax.experimental import pallas as pl
from jax.experimental.pallas import tpu as pltpu
from jax.experimental.pallas import tpu_sc as plsc

N = 10000
E = 320000
D = 128
H = 128
G = 64

NC = 2
NS = 16
NW = NC * NS
E_PER_W = E // NW
CHUNK = 40
N_CHUNKS = E_PER_W // CHUNK
NP_ = 10240
ROWS_PER_TILE = NP_ // NS
ZROWS = 16
ZSEM = 8

BN = 10000


_sc_mesh = plsc.VectorSubcoreMesh(core_axis_name="c", subcore_axis_name="s")


KBUF = 5


@functools.partial(
    pl.kernel,
    out_type=jax.ShapeDtypeStruct((NC * NP_, H), jnp.float32),
    mesh=_sc_mesh,
    scratch_types=[
        pltpu.VMEM((E_PER_W,), jnp.int32),
        pltpu.VMEM((E_PER_W,), jnp.int32),
        [pltpu.VMEM((CHUNK, H), jnp.float32) for _ in range(KBUF)],
        [pltpu.SemaphoreType.DMA for _ in range(KBUF)],
        [pltpu.SemaphoreType.DMA for _ in range(KBUF)],
        [pltpu.SemaphoreType.DMA for _ in range(2)],
        [pltpu.SemaphoreType.DMA for _ in range(ZSEM)],
        pltpu.VMEM((ZROWS, H), jnp.float32),
        pltpu.VMEM_SHARED((NP_, H), jnp.float32),
    ],
)
def _edge_aggregate(y_hbm, src_hbm, dst_hbm, out_hbm,
                    src_v, dst_v, rows_bufs, sems, ssems, isems, zsems,
                    zeros_v, acc_sh):
    c = lax.axis_index("c")
    s = lax.axis_index("s")
    wid = s * NC + c

    ih0 = pltpu.async_copy(src_hbm.at[pl.ds(wid * E_PER_W, E_PER_W)],
                           src_v, isems[0])
    ih1 = pltpu.async_copy(dst_hbm.at[pl.ds(wid * E_PER_W, E_PER_W)],
                           dst_v, isems[1])

    zvec = jnp.zeros((16,), jnp.float32)

    def zrow(i, carry):
        for j in range(H // 16):
            zeros_v[i, pl.ds(j * 16, 16)] = zvec
        return carry

    lax.fori_loop(0, ZROWS, zrow, 0)

    row0 = s * ROWS_PER_TILE
    zh = [pltpu.async_copy(zeros_v, acc_sh.at[pl.ds(row0 + i * ZROWS, ZROWS)],
                           zsems[i % ZSEM])
          for i in range(ROWS_PER_TILE // ZROWS)]
    ih0.wait()
    ih1.wait()
    for h in zh:
        h.wait()

    plsc.subcore_barrier()

    def body(k, carry):
        base = k * KBUF * CHUNK
        gh = []
        for b in range(KBUF):
            gh.append(pltpu.async_copy(
                y_hbm.at[src_v.at[pl.ds(base + b * CHUNK, CHUNK)]],
                rows_bufs[b], sems[b]))
        sh = []
        for b in range(KBUF):
            gh[b].wait()
            sh.append(pltpu.async_copy(
                rows_bufs[b],
                acc_sh.at[dst_v.at[pl.ds(base + b * CHUNK, CHUNK)]],
                ssems[b], add=True))
        for b in range(KBUF):
            sh[b].wait()
        return carry

    lax.fori_loop(0, N_CHUNKS // KBUF, body, 0)

    plsc.subcore_barrier()

    pltpu.sync_copy(acc_sh.at[pl.ds(row0, ROWS_PER_TILE)],
                    out_hbm.at[pl.ds(c * NP_ + row0, ROWS_PER_TILE)])


def _dbl_body(p0_ref, p1_ref, b_ref, w1_ref, w2_ref, o_ref):
    agg = p0_ref[...] + p1_ref[...]
    h = jnp.maximum(
        lax.dot_general(agg, w1_ref[...], (((1,), (1,)), ((), ())),
                        preferred_element_type=jnp.float32) + b_ref[...],
        0.0)
    o_ref[...] = lax.dot_general(
        h, w2_ref[...], (((1,), (1,)), ((), ())),
        preferred_element_type=jnp.float32)


def _double_matmul(p0, p1, b, w1, w2):
    return pl.pallas_call(
        _dbl_body,
        grid=(N // BN,),
        in_specs=[pl.BlockSpec((BN, H), lambda i: (i, 0)),
                  pl.BlockSpec((BN, H), lambda i: (i, 0)),
                  pl.BlockSpec((1, H), lambda i: (0, 0)),
                  pl.BlockSpec((H, D), lambda i: (0, 0)),
                  pl.BlockSpec((H, H), lambda i: (0, 0))],
        out_specs=pl.BlockSpec((BN, H), lambda i: (i, 0)),
        out_shape=jax.ShapeDtypeStruct((N, H), jnp.float32),
    )(p0, p1, b, w1, w2)


def _pool_body(p0_ref, p1_ref, b_ref, batch_ref, o_ref):
    i = pl.program_id(0)
    h2 = jnp.maximum(p0_ref[...] + p1_ref[...] + b_ref[...], 0.0)
    bb = batch_ref[0, 0, :]
    mask = (lax.broadcasted_iota(jnp.int32, (G, BN), 0)
            == bb[None, :]).astype(jnp.float32)
    acc = lax.dot_general(mask, h2, (((1,), (0,)), ((), ())),
                          preferred_element_type=jnp.float32)

    @pl.when(i == 0)
    def _init():
        o_ref[...] = acc

    @pl.when(i > 0)
    def _accum():
        o_ref[...] += acc


def _pool(p0, p1, b, batch3):
    return pl.pallas_call(
        _pool_body,
        grid=(N // BN,),
        in_specs=[pl.BlockSpec((BN, H), lambda i: (i, 0)),
                  pl.BlockSpec((BN, H), lambda i: (i, 0)),
                  pl.BlockSpec((1, H), lambda i: (0, 0)),
                  pl.BlockSpec((1, 1, BN), lambda i: (i, 0, 0))],
        out_specs=pl.BlockSpec((G, H), lambda i: (0, 0)),
        out_shape=jax.ShapeDtypeStruct((G, H), jnp.float32),
    )(p0, p1, b, batch3)


def kernel(x, edge_index, batch, W1, b1, W2, b2):
    src = edge_index[0]
    dst = edge_index[1]

    p1 = _edge_aggregate(x, src, dst)
    y2 = _double_matmul(p1[:N], p1[NP_:NP_ + N], b1.reshape(1, H), W1, W2)
    p2 = _edge_aggregate(y2, src, dst)
    out = _pool(p2[:N], p2[NP_:NP_ + N], b2.reshape(1, H),
                batch.reshape(N // BN, 1, BN))
    return out


# speedup vs baseline: 14.5221x; 1.3928x over previous
import functools

import jax
import jax.numpy as jnp
from jax import lax
from jax.experimental import pallas as pl
from jax.experimental.pallas import tpu as pltpu
from jax.experimental.pallas import tpu_sc as plsc

N = 10000
E = 320000
D = 128
H = 128
G = 64

NC = 2
NS = 16
NW = NC * NS
E_PER_W = E // NW
CHUNK = 40
N_CHUNKS = E_PER_W // CHUNK
NP_ = 10240
ROWS_PER_TILE = NP_ // NS
ZROWS = 16
ZSEM = 8

BN = 10000


_sc_mesh = plsc.VectorSubcoreMesh(core_axis_name="c", subcore_axis_name="s")


KBUF = 5
KG = 3
KS = KBUF - KG


@functools.partial(
    pl.kernel,
    out_type=jax.ShapeDtypeStruct((NC * NP_, H), jnp.float32),
    mesh=_sc_mesh,
    scratch_types=[
        pltpu.VMEM((E_PER_W,), jnp.int32),
        pltpu.VMEM((E_PER_W,), jnp.int32),
        [pltpu.VMEM((CHUNK, H), jnp.float32) for _ in range(KBUF)],
        [pltpu.SemaphoreType.DMA for _ in range(KBUF)],
        [pltpu.SemaphoreType.DMA for _ in range(KBUF)],
        [pltpu.SemaphoreType.DMA for _ in range(2)],
        [pltpu.SemaphoreType.DMA for _ in range(ZSEM)],
        pltpu.VMEM((ZROWS, H), jnp.float32),
        pltpu.VMEM_SHARED((NP_, H), jnp.float32),
    ],
)
def _edge_aggregate(y_hbm, src_hbm, dst_hbm, out_hbm,
                    src_v, dst_v, rows_bufs, sems, ssems, isems, zsems,
                    zeros_v, acc_sh):
    c = lax.axis_index("c")
    s = lax.axis_index("s")
    wid = s * NC + c

    ih0 = pltpu.async_copy(src_hbm.at[pl.ds(wid * E_PER_W, E_PER_W)],
                           src_v, isems[0])
    ih1 = pltpu.async_copy(dst_hbm.at[pl.ds(wid * E_PER_W, E_PER_W)],
                           dst_v, isems[1])

    zvec = jnp.zeros((16,), jnp.float32)

    def zrow(i, carry):
        for j in range(H // 16):
            zeros_v[i, pl.ds(j * 16, 16)] = zvec
        return carry

    lax.fori_loop(0, ZROWS, zrow, 0)

    row0 = s * ROWS_PER_TILE
    zh = [pltpu.async_copy(zeros_v, acc_sh.at[pl.ds(row0 + i * ZROWS, ZROWS)],
                           zsems[i % ZSEM])
          for i in range(ROWS_PER_TILE // ZROWS)]
    ih0.wait()
    ih1.wait()
    for h in zh:
        h.wait()

    plsc.subcore_barrier()

    def fire_g(i, b):
        return pltpu.async_copy(
            y_hbm.at[src_v.at[pl.ds(i * CHUNK, CHUNK)]],
            rows_bufs[b], sems[b])

    def fire_s(i, b):
        return pltpu.async_copy(
            rows_bufs[b],
            acc_sh.at[dst_v.at[pl.ds(i * CHUNK, CHUNK)]],
            ssems[b], add=True)

    def wait_g(b):
        pltpu.make_async_copy(y_hbm.at[pl.ds(0, CHUNK)],
                              rows_bufs[b], sems[b]).wait()

    def wait_s(b):
        pltpu.make_async_copy(y_hbm.at[pl.ds(0, CHUNK)],
                              rows_bufs[b], ssems[b]).wait()

    def round_body(r, first, last):
        base = r * KBUF
        for b in range(KBUF):
            bn = (b + KG) % KBUF
            if not (first and b < KS):
                wait_s(bn)
            if not (last and b >= KS):
                fire_g(base + b + KG, bn)
            wait_g(b)
            fire_s(base + b, b)

    for b in range(KG):
        fire_g(b, b)
    round_body(0, True, False)
    lax.fori_loop(1, N_CHUNKS // KBUF - 1,
                  lambda r, c: (round_body(r, False, False), c)[1], 0)
    round_body(N_CHUNKS // KBUF - 1, False, True)
    for j in range(KS):
        wait_s((N_CHUNKS - KS + j) % KBUF)

    plsc.subcore_barrier()

    pltpu.sync_copy(acc_sh.at[pl.ds(row0, ROWS_PER_TILE)],
                    out_hbm.at[pl.ds(c * NP_ + row0, ROWS_PER_TILE)])


def _dbl_body(p0_ref, p1_ref, b_ref, w1_ref, w2_ref, o_ref):
    agg = p0_ref[...] + p1_ref[...]
    h = jnp.maximum(
        lax.dot_general(agg, w1_ref[...], (((1,), (1,)), ((), ())),
                        preferred_element_type=jnp.float32) + b_ref[...],
        0.0)
    o_ref[...] = lax.dot_general(
        h, w2_ref[...], (((1,), (1,)), ((), ())),
        preferred_element_type=jnp.float32)


def _double_matmul(p0, p1, b, w1, w2):
    return pl.pallas_call(
        _dbl_body,
        grid=(N // BN,),
        in_specs=[pl.BlockSpec((BN, H), lambda i: (i, 0)),
                  pl.BlockSpec((BN, H), lambda i: (i, 0)),
                  pl.BlockSpec((1, H), lambda i: (0, 0)),
                  pl.BlockSpec((H, D), lambda i: (0, 0)),
                  pl.BlockSpec((H, H), lambda i: (0, 0))],
        out_specs=pl.BlockSpec((BN, H), lambda i: (i, 0)),
        out_shape=jax.ShapeDtypeStruct((N, H), jnp.float32),
    )(p0, p1, b, w1, w2)


def _pool_body(p0_ref, p1_ref, b_ref, batch_ref, o_ref):
    i = pl.program_id(0)
    h2 = jnp.maximum(p0_ref[...] + p1_ref[...] + b_ref[...], 0.0)
    bb = batch_ref[0, 0, :]
    mask = (lax.broadcasted_iota(jnp.int32, (G, BN), 0)
            == bb[None, :]).astype(jnp.float32)
    acc = lax.dot_general(mask, h2, (((1,), (0,)), ((), ())),
                          preferred_element_type=jnp.float32)

    @pl.when(i == 0)
    def _init():
        o_ref[...] = acc

    @pl.when(i > 0)
    def _accum():
        o_ref[...] += acc


def _pool(p0, p1, b, batch3):
    return pl.pallas_call(
        _pool_body,
        grid=(N // BN,),
        in_specs=[pl.BlockSpec((BN, H), lambda i: (i, 0)),
                  pl.BlockSpec((BN, H), lambda i: (i, 0)),
                  pl.BlockSpec((1, H), lambda i: (0, 0)),
                  pl.BlockSpec((1, 1, BN), lambda i: (i, 0, 0))],
        out_specs=pl.BlockSpec((G, H), lambda i: (0, 0)),
        out_shape=jax.ShapeDtypeStruct((G, H), jnp.float32),
    )(p0, p1, b, batch3)


def kernel(x, edge_index, batch, W1, b1, W2, b2):
    src = edge_index[0]
    dst = edge_index[1]

    p1 = _edge_aggregate(x, src, dst)
    y2 = _double_matmul(p1[:N], p1[NP_:NP_ + N], b1.reshape(1, H), W1, W2)
    p2 = _edge_aggregate(y2, src, dst)
    out = _pool(p2[:N], p2[NP_:NP_ + N], b2.reshape(1, H),
                batch.reshape(N // BN, 1, BN))
    return out
